# baseline (speedup 1.0000x reference)
import functools

import jax
import jax.numpy as jnp
from jax import lax
from jax.experimental import pallas as pl
from jax.experimental.pallas import tpu as pltpu
from jax.experimental.pallas import tpu_sc as plsc

_NW = 32
_CHUNK = 64


def _sc_gather(table, idx):
    n, d = idx.shape[0], table.shape[1]
    b_per_w = n // _NW
    n_chunks = b_per_w // _CHUNK
    mesh = plsc.VectorSubcoreMesh(core_axis_name="c", subcore_axis_name="s")

    @functools.partial(
        pl.kernel,
        mesh=mesh,
        out_type=jax.ShapeDtypeStruct((n, d), jnp.float32),
        scratch_types=[
            pltpu.VMEM((b_per_w,), jnp.int32),
            pltpu.VMEM((2, _CHUNK, d), jnp.float32),
            pltpu.SemaphoreType.DMA,
        ],
    )
    def k(table_hbm, idx_hbm, out_hbm, idx_v, rows_v, gsem):
        wid = lax.axis_index("s") * 2 + lax.axis_index("c")
        base = wid * b_per_w
        pltpu.sync_copy(idx_hbm.at[pl.ds(base, b_per_w)], idx_v)
        pltpu.async_copy(
            table_hbm.at[idx_v.at[pl.ds(0, _CHUNK)]], rows_v.at[0], gsem)

        def body(kk, _):
            slot = lax.rem(kk, 2)
            nxt = lax.rem(kk + 1, 2)
            pltpu.make_async_copy(
                table_hbm.at[idx_v.at[pl.ds(0, _CHUNK)]],
                rows_v.at[slot], gsem).wait()

            @pl.when(kk + 1 < n_chunks)
            def _start_next():
                pltpu.async_copy(
                    table_hbm.at[idx_v.at[pl.ds((kk + 1) * _CHUNK, _CHUNK)]],
                    rows_v.at[nxt], gsem)

            pltpu.sync_copy(
                rows_v.at[slot], out_hbm.at[pl.ds(base + kk * _CHUNK, _CHUNK)])
            return 0

        lax.fori_loop(0, n_chunks, body, 0)

    return k(table, idx)


def _tc_addln_slice(prev, words, pos_emb, type_pad, tids3, gamma2, beta2,
                    n_total, blk0):
    n, d = words.shape
    seq = pos_emb.shape[0]
    t_blk = 256
    grid = n // t_blk
    blocks_per_seq = seq // t_blk

    def body(*refs):
        if prev is None:
            w_ref, p_ref, t_ref, id_ref, g_ref, b_ref, o_ref = refs
        else:
            _, w_ref, p_ref, t_ref, id_ref, g_ref, b_ref, o_ref = refs
        tid_col = id_ref[...]
        delta = t_ref[1:2, :] - t_ref[0:1, :]
        v = w_ref[...] + p_ref[...] + t_ref[0:1, :] + tid_col * delta
        mean = jnp.mean(v, axis=1, keepdims=True)
        c = v - mean
        var = jnp.mean(c * c, axis=1, keepdims=True)
        o_ref[...] = c * lax.rsqrt(var + 1e-12) * g_ref[...] + b_ref[...]

    in_specs = [
        pl.BlockSpec((t_blk, d), lambda i: (i, 0)),
        pl.BlockSpec((t_blk, d), lambda i: (i % blocks_per_seq, 0)),
        pl.BlockSpec((8, d), lambda i: (0, 0)),
        pl.BlockSpec((t_blk, 1), lambda i: (i, 0)),
        pl.BlockSpec((1, d), lambda i: (0, 0)),
        pl.BlockSpec((1, d), lambda i: (0, 0)),
    ]
    args = (words, pos_emb, type_pad, tids3, gamma2, beta2)
    aliases = {}
    if prev is not None:
        in_specs = [pl.BlockSpec(memory_space=pl.ANY)] + in_specs
        args = (prev,) + args
        aliases = {0: 0}
    return pl.pallas_call(
        body,
        grid=(grid,),
        in_specs=in_specs,
        out_specs=pl.BlockSpec((t_blk, d), lambda i: (blk0 + i, 0)),
        out_shape=jax.ShapeDtypeStruct((n_total, d), jnp.float32),
        input_output_aliases=aliases,
    )(*args)


def kernel(input_ids, token_type_ids, attention_mask, word_embeddings,
           position_embeddings, token_type_embeddings, ln_gamma, ln_beta):
    b, l = input_ids.shape
    d = word_embeddings.shape[1]
    n = b * l
    n_slices = 4
    ns = n // n_slices
    t_blk = 256
    ids_flat = input_ids.reshape(-1).astype(jnp.int32)
    tids_col = token_type_ids.reshape(-1, 1).astype(jnp.float32)
    type_pad = jnp.zeros((8, d), jnp.float32).at[:2].set(token_type_embeddings)
    gamma2 = ln_gamma.reshape(1, d)
    beta2 = ln_beta.reshape(1, d)

    word_slices = [
        _sc_gather(word_embeddings, lax.slice(ids_flat, (i * ns,), ((i + 1) * ns,)))
        for i in range(n_slices)
    ]
    out = None
    for i in range(n_slices):
        tslice = lax.slice(tids_col, (i * ns, 0), ((i + 1) * ns, 1))
        out = _tc_addln_slice(out, word_slices[i], position_embeddings,
                              type_pad, tslice, gamma2, beta2,
                              n_total=n, blk0=i * (ns // t_blk))
    return (out.reshape(b, l, d), attention_mask)


# speedup vs baseline: 1.1804x; 1.1804x over previous
import functools

import jax
import jax.numpy as jnp
from jax import lax
from jax.experimental import pallas as pl
from jax.experimental.pallas import tpu as pltpu
from jax.experimental.pallas import tpu_sc as plsc

_NW = 32
_CHUNK = 64


def _sc_gather(table, idx):
    n, d = idx.shape[0], table.shape[1]
    b_per_w = n // _NW
    n_chunks = b_per_w // _CHUNK
    mesh = plsc.VectorSubcoreMesh(core_axis_name="c", subcore_axis_name="s")

    @functools.partial(
        pl.kernel,
        mesh=mesh,
        out_type=jax.ShapeDtypeStruct((n, d), jnp.float32),
        scratch_types=[
            pltpu.VMEM((b_per_w,), jnp.int32),
            pltpu.VMEM((2, _CHUNK, d), jnp.float32),
            pltpu.SemaphoreType.DMA,
        ],
    )
    def k(table_hbm, idx_hbm, out_hbm, idx_v, rows_v, gsem):
        wid = lax.axis_index("s") * 2 + lax.axis_index("c")
        base = wid * b_per_w
        pltpu.sync_copy(idx_hbm.at[pl.ds(base, b_per_w)], idx_v)
        pltpu.async_copy(
            table_hbm.at[idx_v.at[pl.ds(0, _CHUNK)]], rows_v.at[0], gsem)

        def body(kk, _):
            slot = lax.rem(kk, 2)
            nxt = lax.rem(kk + 1, 2)
            pltpu.make_async_copy(
                table_hbm.at[idx_v.at[pl.ds(0, _CHUNK)]],
                rows_v.at[slot], gsem).wait()

            @pl.when(kk + 1 < n_chunks)
            def _start_next():
                pltpu.async_copy(
                    table_hbm.at[idx_v.at[pl.ds((kk + 1) * _CHUNK, _CHUNK)]],
                    rows_v.at[nxt], gsem)

            pltpu.sync_copy(
                rows_v.at[slot], out_hbm.at[pl.ds(base + kk * _CHUNK, _CHUNK)])
            return 0

        lax.fori_loop(0, n_chunks, body, 0)

    return k(table, idx)


def _tc_addln_slice(prev, words, pos_emb, type_pad, tids3, gamma2, beta2,
                    n_total, blk0):
    n, d = words.shape
    seq = pos_emb.shape[0]
    t_blk = 256
    grid = n // t_blk
    blocks_per_seq = seq // t_blk

    def body(*refs):
        if prev is None:
            w_ref, p_ref, t_ref, id_ref, g_ref, b_ref, o_ref = refs
        else:
            _, w_ref, p_ref, t_ref, id_ref, g_ref, b_ref, o_ref = refs
        i = pl.program_id(0)
        p = p_ref[pl.ds(lax.rem(i, blocks_per_seq) * t_blk, t_blk), :]
        ids2 = id_ref[0].astype(jnp.float32)
        r_io = lax.broadcasted_iota(jnp.int32, (t_blk, t_blk), 0)
        c_io = lax.broadcasted_iota(jnp.int32, (t_blk, t_blk), 1)
        a = (r_io == c_io).astype(jnp.float32) * ids2
        delta = jnp.broadcast_to(t_ref[1:2, :] - t_ref[0:1, :], (t_blk, d))
        tv = jnp.dot(a, delta, preferred_element_type=jnp.float32)
        v = w_ref[...] + p + t_ref[0:1, :] + tv
        mean = jnp.mean(v, axis=1, keepdims=True)
        c = v - mean
        var = jnp.mean(c * c, axis=1, keepdims=True)
        o_ref[...] = c * lax.rsqrt(var + 1e-12) * g_ref[...] + b_ref[...]

    in_specs = [
        pl.BlockSpec((t_blk, d), lambda i: (i, 0)),
        pl.BlockSpec((seq, d), lambda i: (0, 0)),
        pl.BlockSpec((8, d), lambda i: (0, 0)),
        pl.BlockSpec((1, 1, t_blk), lambda i: (i, 0, 0)),
        pl.BlockSpec((1, d), lambda i: (0, 0)),
        pl.BlockSpec((1, d), lambda i: (0, 0)),
    ]
    args = (words, pos_emb, type_pad, tids3, gamma2, beta2)
    aliases = {}
    if prev is not None:
        in_specs = [pl.BlockSpec(memory_space=pl.ANY)] + in_specs
        args = (prev,) + args
        aliases = {0: 0}
    return pl.pallas_call(
        body,
        grid=(grid,),
        in_specs=in_specs,
        out_specs=pl.BlockSpec((t_blk, d), lambda i: (blk0 + i, 0)),
        out_shape=jax.ShapeDtypeStruct((n_total, d), jnp.float32),
        input_output_aliases=aliases,
    )(*args)


def kernel(input_ids, token_type_ids, attention_mask, word_embeddings,
           position_embeddings, token_type_embeddings, ln_gamma, ln_beta):
    b, l = input_ids.shape
    d = word_embeddings.shape[1]
    n = b * l
    n_slices = 4
    ns = n // n_slices
    t_blk = 256
    ids_flat = input_ids.reshape(-1).astype(jnp.int32)
    tids_flat = token_type_ids.reshape(-1).astype(jnp.int32)
    type_pad = jnp.zeros((8, d), jnp.float32).at[:2].set(token_type_embeddings)
    gamma2 = ln_gamma.reshape(1, d)
    beta2 = ln_beta.reshape(1, d)

    word_slices = [
        _sc_gather(word_embeddings, lax.slice(ids_flat, (i * ns,), ((i + 1) * ns,)))
        for i in range(n_slices)
    ]
    out = None
    for i in range(n_slices):
        tids3 = lax.slice(tids_flat, (i * ns,), ((i + 1) * ns,)).reshape(
            ns // t_blk, 1, t_blk)
        out = _tc_addln_slice(out, word_slices[i], position_embeddings,
                              type_pad, tids3, gamma2, beta2,
                              n_total=n, blk0=i * (ns // t_blk))
    return (out.reshape(b, l, d), attention_mask)


# speedup vs baseline: 1.3714x; 1.1618x over previous
import functools

import jax
import jax.numpy as jnp
from jax import lax
from jax.experimental import pallas as pl
from jax.experimental.pallas import tpu as pltpu
from jax.experimental.pallas import tpu_sc as plsc

_NW = 32
_CHUNK = 64


def _sc_gather(table, idx, off, n):
    d = table.shape[1]
    b_per_w = n // _NW
    n_chunks = b_per_w // _CHUNK
    mesh = plsc.VectorSubcoreMesh(core_axis_name="c", subcore_axis_name="s")

    @functools.partial(
        pl.kernel,
        mesh=mesh,
        out_type=jax.ShapeDtypeStruct((n, d), jnp.float32),
        scratch_types=[
            pltpu.VMEM((b_per_w,), jnp.int32),
            pltpu.VMEM((2, _CHUNK, d), jnp.float32),
            pltpu.SemaphoreType.DMA,
        ],
    )
    def k(table_hbm, idx_hbm, out_hbm, idx_v, rows_v, gsem):
        wid = lax.axis_index("s") * 2 + lax.axis_index("c")
        base = wid * b_per_w
        pltpu.sync_copy(idx_hbm.at[pl.ds(off + base, b_per_w)], idx_v)
        pltpu.async_copy(
            table_hbm.at[idx_v.at[pl.ds(0, _CHUNK)]], rows_v.at[0], gsem)

        def body(kk, _):
            slot = lax.rem(kk, 2)
            nxt = lax.rem(kk + 1, 2)
            pltpu.make_async_copy(
                table_hbm.at[idx_v.at[pl.ds(0, _CHUNK)]],
                rows_v.at[slot], gsem).wait()

            @pl.when(kk + 1 < n_chunks)
            def _start_next():
                pltpu.async_copy(
                    table_hbm.at[idx_v.at[pl.ds((kk + 1) * _CHUNK, _CHUNK)]],
                    rows_v.at[nxt], gsem)

            pltpu.sync_copy(
                rows_v.at[slot], out_hbm.at[pl.ds(base + kk * _CHUNK, _CHUNK)])
            return 0

        lax.fori_loop(0, n_chunks, body, 0)

    return k(table, idx)


def _tc_addln_slice(prev, words, pos_emb, type_pad, tids3, gamma2, beta2,
                    n_total, blk0):
    n, d = words.shape
    seq = pos_emb.shape[0]
    t_blk = 512
    grid = n // t_blk
    blocks_per_seq = max(seq // t_blk, 1)

    def body(*refs):
        if prev is None:
            w_ref, p_ref, t_ref, id_ref, g_ref, b_ref, o_ref = refs
        else:
            _, w_ref, p_ref, t_ref, id_ref, g_ref, b_ref, o_ref = refs
        i = pl.program_id(0)
        p = p_ref[pl.ds(lax.rem(i, blocks_per_seq) * min(t_blk, seq), t_blk), :]
        ids2 = id_ref[0].astype(jnp.bfloat16)
        r_io = lax.broadcasted_iota(jnp.int32, (t_blk, t_blk), 0)
        c_io = lax.broadcasted_iota(jnp.int32, (t_blk, t_blk), 1)
        a = (r_io == c_io).astype(jnp.bfloat16) * ids2
        delta = jnp.broadcast_to(
            (t_ref[1:2, :] - t_ref[0:1, :]).astype(jnp.bfloat16), (t_blk, d))
        tv = jnp.dot(a, delta, preferred_element_type=jnp.float32)
        v = w_ref[...] + p + t_ref[0:1, :] + tv
        mean = jnp.mean(v, axis=1, keepdims=True)
        c = v - mean
        var = jnp.mean(c * c, axis=1, keepdims=True)
        o_ref[...] = c * lax.rsqrt(var + 1e-12) * g_ref[...] + b_ref[...]

    in_specs = [
        pl.BlockSpec((t_blk, d), lambda i: (i, 0)),
        pl.BlockSpec((seq, d), lambda i: (0, 0)),
        pl.BlockSpec((8, d), lambda i: (0, 0)),
        pl.BlockSpec((1, 1, t_blk), lambda i: (blk0 + i, 0, 0)),
        pl.BlockSpec((1, d), lambda i: (0, 0)),
        pl.BlockSpec((1, d), lambda i: (0, 0)),
    ]
    args = (words, pos_emb, type_pad, tids3, gamma2, beta2)
    aliases = {}
    if prev is not None:
        in_specs = [pl.BlockSpec(memory_space=pl.ANY)] + in_specs
        args = (prev,) + args
        aliases = {0: 0}
    return pl.pallas_call(
        body,
        grid=(grid,),
        in_specs=in_specs,
        out_specs=pl.BlockSpec((t_blk, d), lambda i: (blk0 + i, 0)),
        out_shape=jax.ShapeDtypeStruct((n_total, d), jnp.float32),
        input_output_aliases=aliases,
    )(*args)


def kernel(input_ids, token_type_ids, attention_mask, word_embeddings,
           position_embeddings, token_type_embeddings, ln_gamma, ln_beta):
    b, l = input_ids.shape
    d = word_embeddings.shape[1]
    n = b * l
    n_slices = 4
    ns = n // n_slices
    t_blk = 512
    ids_flat = input_ids.reshape(-1).astype(jnp.int32)
    tids3 = token_type_ids.reshape(n // t_blk, 1, t_blk).astype(jnp.int32)
    type_pad = jnp.zeros((8, d), jnp.float32).at[:2].set(token_type_embeddings)
    gamma2 = ln_gamma.reshape(1, d)
    beta2 = ln_beta.reshape(1, d)

    word_slices = [
        _sc_gather(word_embeddings, ids_flat, i * ns, ns)
        for i in range(n_slices)
    ]
    out = None
    for i in range(n_slices):
        out = _tc_addln_slice(out, word_slices[i], position_embeddings,
                              type_pad, tids3, gamma2, beta2,
                              n_total=n, blk0=i * (ns // t_blk))
    return (out.reshape(b, l, d), attention_mask)
